# baseline (speedup 1.0000x reference)
import jax
import jax.numpy as jnp
from jax.experimental import pallas as pl
from jax.experimental.pallas import tpu as pltpu

EPS = 1e-5


def _encoder_kernel(x_ref, mf_ref, w1_ref, b1_ref, w2_ref, b2_ref,
                    w3a_ref, w3b_ref, b3_ref, w4_ref, b4_ref,
                    out_ref, feat_scr, pooled_scr):
    phase = pl.program_id(1)

    @pl.when(phase == 0)
    def _():
        xb = x_ref[0]
        mf = mf_ref[0]
        h = jnp.dot(xb, w1_ref[...], preferred_element_type=jnp.float32)
        h = jnp.maximum(h + b1_ref[...], 0.0)
        feat = jnp.dot(h, w2_ref[...], preferred_element_type=jnp.float32)
        feat = feat + b2_ref[...]
        fm = jnp.where(mf != 0.0, feat, 0.0)
        feat_scr[...] = fm
        pooled_scr[...] = jnp.max(fm, axis=0, keepdims=True)

    @pl.when(phase == 1)
    def _():
        mf = mf_ref[0]
        fm = feat_scr[...]
        pc = jnp.dot(pooled_scr[...], w3b_ref[...],
                     preferred_element_type=jnp.float32)
        h2 = jnp.dot(fm, w3a_ref[...], preferred_element_type=jnp.float32)
        h2 = jnp.maximum(h2 + pc + b3_ref[...], 0.0)
        op = jnp.dot(h2, w4_ref[...], preferred_element_type=jnp.float32)
        op = op + b4_ref[...]
        op = jnp.where(mf != 0.0, op, 0.0)
        out_ref[0] = jnp.max(op, axis=0, keepdims=True)


def kernel(x, mask, W1, b1, g1, be1, W2, b2, W3, b3, g2, be2, W4, b4):
    B, M, C = x.shape
    EC = W4.shape[1]

    s1 = g1 / jnp.sqrt(1.0 + EPS)
    W1f = W1 * s1[None, :]
    b1f = (b1 * s1 + be1)[None, :]
    s2 = g2 / jnp.sqrt(1.0 + EPS)
    W3s = W3 * s2[None, :]
    W3a = W3s[:256]
    W3b = W3s[256:]
    b3f = (b3 * s2 + be2)[None, :]
    b2r = b2[None, :]
    b4r = b4[None, :]

    mf = mask.astype(jnp.float32)[..., None]

    out = pl.pallas_call(
        _encoder_kernel,
        grid=(B, 2),
        in_specs=[
            pl.BlockSpec((1, M, C), lambda b, p: (b, 0, 0)),
            pl.BlockSpec((1, M, 1), lambda b, p: (b, 0, 0)),
            pl.BlockSpec((C, 128), lambda b, p: (0, 0)),
            pl.BlockSpec((1, 128), lambda b, p: (0, 0)),
            pl.BlockSpec((128, 256), lambda b, p: (0, 0)),
            pl.BlockSpec((1, 256), lambda b, p: (0, 0)),
            pl.BlockSpec((256, 256), lambda b, p: (0, 0)),
            pl.BlockSpec((256, 256), lambda b, p: (0, 0)),
            pl.BlockSpec((1, 256), lambda b, p: (0, 0)),
            pl.BlockSpec((256, EC), lambda b, p: (0, 0)),
            pl.BlockSpec((1, EC), lambda b, p: (0, 0)),
        ],
        out_specs=pl.BlockSpec((1, 1, EC), lambda b, p: (b, 0, 0)),
        out_shape=jax.ShapeDtypeStruct((B, 1, EC), jnp.float32),
        scratch_shapes=[
            pltpu.VMEM((M, 256), jnp.float32),
            pltpu.VMEM((1, 256), jnp.float32),
        ],
    )(x, mf, W1f, b1f, W2, b2r, W3a, W3b, b3f, W4, b4r)
    return out.reshape(B, EC)
